# Initial kernel scaffold; baseline (speedup 1.0000x reference)
import functools

import jax
import jax.numpy as jnp
from jax import lax
from jax.experimental import pallas as pl
from jax.experimental.pallas import tpu as pltpu
from jax.experimental.pallas import tpu_sc as plsc

ALPHA = 0.01
K = 10

N = 10000
E = 320000
D = 128

NC = 2
NS = 16
L = 16
NW = NC * NS

G = 80
EPT = E // NW
NCH = EPT // G
RPT = N // NS

_vector_mesh = plsc.VectorSubcoreMesh(core_axis_name="c", subcore_axis_name="s")


def _mlp_body(x_ref, w1_ref, b1_ref, w2_ref, b2_ref, o_ref):
    h = lax.dot_general(x_ref[...], w1_ref[...], (((1,), (1,)), ((), ())),
                        preferred_element_type=jnp.float32,
                        precision=lax.Precision.HIGHEST)
    h = jnp.maximum(h + b1_ref[...], 0.0)
    o = lax.dot_general(h, w2_ref[...], (((1,), (1,)), ((), ())),
                        preferred_element_type=jnp.float32,
                        precision=lax.Precision.HIGHEST)
    o_ref[...] = o + b2_ref[...]


def _mlp(x, W1, b1, W2, b2):
    bm = 2000
    return pl.pallas_call(
        _mlp_body,
        grid=(N // bm,),
        in_specs=[
            pl.BlockSpec((bm, D), lambda i: (i, 0)),
            pl.BlockSpec((D, D), lambda i: (0, 0)),
            pl.BlockSpec((1, D), lambda i: (0, 0)),
            pl.BlockSpec((D, D), lambda i: (0, 0)),
            pl.BlockSpec((1, D), lambda i: (0, 0)),
        ],
        out_specs=pl.BlockSpec((bm, D), lambda i: (i, 0)),
        out_shape=jax.ShapeDtypeStruct((N, D), jnp.float32),
    )(x, W1, b1.reshape(1, D), W2, b2.reshape(1, D))


def _combine_body(p_ref, h_ref, o_ref):
    o_ref[...] = ((1.0 - ALPHA) * (p_ref[0] + p_ref[1])
                  + ALPHA * h_ref[...])


def _combine(part, h):
    bm = 2000
    return pl.pallas_call(
        _combine_body,
        grid=(N // bm,),
        in_specs=[
            pl.BlockSpec((NC, bm, D), lambda i: (0, i, 0)),
            pl.BlockSpec((bm, D), lambda i: (i, 0)),
        ],
        out_specs=pl.BlockSpec((bm, D), lambda i: (i, 0)),
        out_shape=jax.ShapeDtypeStruct((N, D), jnp.float32),
    )(part, h)


@functools.partial(
    pl.kernel,
    mesh=_vector_mesh,
    out_type=jax.ShapeDtypeStruct((NC, N, D), jnp.float32),
    scratch_types=[
        pltpu.VMEM((NCH, G), jnp.int32),
        pltpu.VMEM((NCH, G), jnp.int32),
        pltpu.VMEM((EPT,), jnp.float32),
        pltpu.VMEM((G, D), jnp.float32),
        pltpu.VMEM_SHARED((N, D), jnp.float32),
        pltpu.SemaphoreType.DMA,
    ],
)
def _sc_prop(h_hbm, row_hbm, col_hbm, val_hbm, zero_hbm, out_hbm,
             rowv, colv, valv, rowsv, agg, sem):
    c = lax.axis_index("c")
    s = lax.axis_index("s")
    wid = c * NS + s

    pltpu.sync_copy(row_hbm.at[wid], rowv)
    pltpu.sync_copy(col_hbm.at[wid], colv)
    pltpu.sync_copy(val_hbm.at[wid], valv)
    pltpu.sync_copy(zero_hbm.at[pl.ds(s * RPT, RPT)],
                    agg.at[pl.ds(s * RPT, RPT)])
    plsc.subcore_barrier()

    @pl.loop(0, NCH)
    def _chunk(i):
        pltpu.async_copy(h_hbm.at[colv.at[i]], rowsv, sem).wait()

        @pl.loop(0, G)
        def _edge(e):
            vv = plsc.load_gather(valv, [jnp.full((L,), i * G + e, jnp.int32)])

            @pl.loop(0, D, step=L)
            def _feat(j):
                rowsv[e, pl.ds(j, L)] = rowsv[e, pl.ds(j, L)] * vv

        pltpu.sync_copy(rowsv, agg.at[rowv.at[i]], add=True)

    plsc.subcore_barrier()
    pltpu.sync_copy(agg.at[pl.ds(s * RPT, RPT)],
                    out_hbm.at[c, pl.ds(s * RPT, RPT)])


def kernel(x, adj_indices, adj_values, W1, b1, W2, b2):
    h = _mlp(x, W1, b1, W2, b2)
    row = adj_indices[0].reshape(NW, NCH, G)
    col = adj_indices[1].reshape(NW, NCH, G)
    vals = adj_values.reshape(NW, EPT)
    zeros = jnp.zeros((N, D), jnp.float32)
    for _ in range(K):
        part = _sc_prop(h, row, col, vals, zeros)
        h = _combine(part, h)
    return h


# speedup vs baseline: 3.5322x; 3.5322x over previous
import dataclasses
import functools

import jax
import jax.numpy as jnp
from jax import lax
from jax.experimental import pallas as pl
from jax.experimental.pallas import tpu as pltpu
from jax.experimental.pallas import tpu_sc as plsc

ALPHA = 0.01
K = 10

N = 10000
E = 320000
D = 128

NC = 2
NS = 16
L = 16
DH = D // NC

NP = 10240
G = 80
EPT = E // NS
NCH = EPT // G
RPT = NP // NS

_vector_mesh = plsc.VectorSubcoreMesh(core_axis_name="c", subcore_axis_name="s")

_sc_params = pltpu.CompilerParams()
for _f, _v in (("needs_layout_passes", False), ("use_tc_tiling_on_sc", False)):
    if _f in pltpu.CompilerParams.__dataclass_fields__:
        _sc_params = dataclasses.replace(_sc_params, **{_f: _v})


def _mlp_body(x_ref, w1_ref, b1_ref, w2_ref, b2_ref, o_ref):
    h = lax.dot_general(x_ref[...], w1_ref[...], (((1,), (1,)), ((), ())),
                        preferred_element_type=jnp.float32,
                        precision=lax.Precision.HIGHEST)
    h = jnp.maximum(h + b1_ref[...], 0.0)
    o = lax.dot_general(h, w2_ref[...], (((1,), (1,)), ((), ())),
                        preferred_element_type=jnp.float32,
                        precision=lax.Precision.HIGHEST)
    o_ref[...] = o + b2_ref[...]


def _mlp(x, W1, b1, W2, b2):
    bm = 2000
    return pl.pallas_call(
        _mlp_body,
        grid=(N // bm,),
        in_specs=[
            pl.BlockSpec((bm, D), lambda i: (i, 0)),
            pl.BlockSpec((D, D), lambda i: (0, 0)),
            pl.BlockSpec((1, D), lambda i: (0, 0)),
            pl.BlockSpec((D, D), lambda i: (0, 0)),
            pl.BlockSpec((1, D), lambda i: (0, 0)),
        ],
        out_specs=pl.BlockSpec((bm, D), lambda i: (i, 0)),
        out_shape=jax.ShapeDtypeStruct((N, D), jnp.float32),
    )(x, W1, b1.reshape(1, D), W2, b2.reshape(1, D))


def _blend_body(p_ref, h_ref, o_ref):
    o_ref[...] = (1.0 - ALPHA) * p_ref[...] + ALPHA * h_ref[...]


def _blend(part, hs):
    bm = 2048
    spec = pl.BlockSpec((NC, bm, DH), lambda i: (0, i, 0))
    return pl.pallas_call(
        _blend_body,
        grid=(NP // bm,),
        in_specs=[spec, spec],
        out_specs=spec,
        out_shape=jax.ShapeDtypeStruct((NC, NP, DH), jnp.float32),
    )(part, hs)


@functools.partial(
    pl.kernel,
    mesh=_vector_mesh,
    out_type=jax.ShapeDtypeStruct((NC, NP, DH), jnp.float32),
    scratch_types=[
        pltpu.VMEM((NCH, G), jnp.int32),
        pltpu.VMEM((NCH, G), jnp.int32),
        pltpu.VMEM((NCH, G), jnp.float32),
        pltpu.VMEM((G, DH), jnp.float32),
        pltpu.VMEM_SHARED((NP, DH), jnp.float32),
        pltpu.SemaphoreType.DMA,
    ],
    compiler_params=_sc_params,
)
def _sc_prop(hs_hbm, row_hbm, col_hbm, val_hbm, zero_hbm, out_hbm,
             rowv, colv, valv, rowsv, agg, sem):
    c = lax.axis_index("c")
    s = lax.axis_index("s")

    pltpu.sync_copy(row_hbm.at[s], rowv)
    pltpu.sync_copy(col_hbm.at[s], colv)
    pltpu.sync_copy(val_hbm.at[s], valv)
    pltpu.sync_copy(zero_hbm.at[pl.ds(s * RPT, RPT)],
                    agg.at[pl.ds(s * RPT, RPT)])
    plsc.subcore_barrier()

    @pl.loop(0, NCH)
    def _chunk(i):
        pltpu.async_copy(hs_hbm.at[c].at[colv.at[i]], rowsv, sem).wait()

        @pl.loop(0, G)
        def _edge(e):
            vv = plsc.load_gather(
                valv, [jnp.full((L,), i, jnp.int32),
                       jnp.full((L,), e, jnp.int32)])

            @pl.loop(0, DH, step=L)
            def _feat(j):
                rowsv[e, pl.ds(j, L)] = rowsv[e, pl.ds(j, L)] * vv

        pltpu.sync_copy(rowsv, agg.at[rowv.at[i]], add=True)

    plsc.subcore_barrier()
    pltpu.sync_copy(agg.at[pl.ds(s * RPT, RPT)],
                    out_hbm.at[c, pl.ds(s * RPT, RPT)])


def kernel(x, adj_indices, adj_values, W1, b1, W2, b2):
    h = _mlp(x, W1, b1, W2, b2)
    hs = jnp.zeros((NC, NP, DH), jnp.float32)
    hs = hs.at[:, :N, :].set(
        jnp.transpose(h.reshape(N, NC, DH), (1, 0, 2)))
    row = adj_indices[0].reshape(NS, NCH, G)
    col = adj_indices[1].reshape(NS, NCH, G)
    vals = adj_values.reshape(NS, NCH, G)
    zeros = jnp.zeros((NP, DH), jnp.float32)
    for _ in range(K):
        part = _sc_prop(hs, row, col, vals, zeros)
        hs = _blend(part, hs)
    return jnp.transpose(hs[:, :N, :], (1, 0, 2)).reshape(N, D)


# speedup vs baseline: 3.7034x; 1.0485x over previous
import dataclasses
import functools

import jax
import jax.numpy as jnp
from jax import lax
from jax.experimental import pallas as pl
from jax.experimental.pallas import tpu as pltpu
from jax.experimental.pallas import tpu_sc as plsc

ALPHA = 0.01
K = 10

N = 10000
E = 320000
D = 128

NC = 2
NS = 16
L = 16
DH = D // NC

NP = 10240
G = 80
EPT = E // NS
NCH = EPT // G
RPT = NP // NS

_vector_mesh = plsc.VectorSubcoreMesh(core_axis_name="c", subcore_axis_name="s")

_sc_params = pltpu.CompilerParams()
for _f, _v in (("needs_layout_passes", False), ("use_tc_tiling_on_sc", False)):
    if _f in pltpu.CompilerParams.__dataclass_fields__:
        _sc_params = dataclasses.replace(_sc_params, **{_f: _v})


def _mlp_body(x_ref, w1_ref, b1_ref, w2_ref, b2_ref, o_ref):
    h = lax.dot_general(x_ref[...], w1_ref[...], (((1,), (1,)), ((), ())),
                        preferred_element_type=jnp.float32,
                        precision=lax.Precision.HIGHEST)
    h = jnp.maximum(h + b1_ref[...], 0.0)
    o = lax.dot_general(h, w2_ref[...], (((1,), (1,)), ((), ())),
                        preferred_element_type=jnp.float32,
                        precision=lax.Precision.HIGHEST)
    o_ref[...] = o + b2_ref[...]


def _mlp(x, W1, b1, W2, b2):
    bm = 2000
    return pl.pallas_call(
        _mlp_body,
        grid=(N // bm,),
        in_specs=[
            pl.BlockSpec((bm, D), lambda i: (i, 0)),
            pl.BlockSpec((D, D), lambda i: (0, 0)),
            pl.BlockSpec((1, D), lambda i: (0, 0)),
            pl.BlockSpec((D, D), lambda i: (0, 0)),
            pl.BlockSpec((1, D), lambda i: (0, 0)),
        ],
        out_specs=pl.BlockSpec((bm, D), lambda i: (i, 0)),
        out_shape=jax.ShapeDtypeStruct((N, D), jnp.float32),
    )(x, W1, b1.reshape(1, D), W2, b2.reshape(1, D))


def _blend_body(p_ref, h_ref, o_ref):
    o_ref[...] = (1.0 - ALPHA) * p_ref[...] + ALPHA * h_ref[...]


def _blend(part, hs):
    bm = 2048
    spec = pl.BlockSpec((NC, bm, DH), lambda i: (0, i, 0))
    return pl.pallas_call(
        _blend_body,
        grid=(NP // bm,),
        in_specs=[spec, spec],
        out_specs=spec,
        out_shape=jax.ShapeDtypeStruct((NC, NP, DH), jnp.float32),
    )(part, hs)


@functools.partial(
    pl.kernel,
    mesh=_vector_mesh,
    out_type=jax.ShapeDtypeStruct((NC, NP, DH), jnp.float32),
    scratch_types=[
        pltpu.VMEM((NCH, G), jnp.int32),
        pltpu.VMEM((NCH, G), jnp.int32),
        pltpu.VMEM((NCH, G), jnp.float32),
        pltpu.VMEM((G, DH), jnp.float32),
        pltpu.VMEM((G, DH), jnp.float32),
        pltpu.VMEM((G, DH), jnp.float32),
        pltpu.VMEM((G, DH), jnp.float32),
        pltpu.VMEM_SHARED((NP, DH), jnp.float32),
        pltpu.SemaphoreType.DMA,
        pltpu.SemaphoreType.DMA,
        pltpu.SemaphoreType.DMA,
        pltpu.SemaphoreType.DMA,
    ],
    compiler_params=_sc_params,
)
def _sc_prop(hs_hbm, row_hbm, col_hbm, val_hbm, zero_hbm, out_hbm,
             rowv, colv, valv, gbuf0, gbuf1, sbuf0, sbuf1, agg,
             gsem0, gsem1, ssem0, ssem1):
    c = lax.axis_index("c")
    s = lax.axis_index("s")
    gbufs = (gbuf0, gbuf1)
    sbufs = (sbuf0, sbuf1)
    gsems = (gsem0, gsem1)
    ssems = (ssem0, ssem1)

    pltpu.sync_copy(row_hbm.at[s], rowv)
    pltpu.sync_copy(col_hbm.at[s], colv)
    pltpu.sync_copy(val_hbm.at[s], valv)
    pltpu.sync_copy(zero_hbm.at[pl.ds(s * RPT, RPT)],
                    agg.at[pl.ds(s * RPT, RPT)])
    plsc.subcore_barrier()

    pltpu.async_copy(hs_hbm.at[c].at[colv.at[0]], gbuf0, gsem0)
    pltpu.async_copy(hs_hbm.at[c].at[colv.at[1]], gbuf1, gsem1)

    @pl.loop(0, NCH // 2)
    def _pair(t):
        for b in range(2):
            i = 2 * t + b

            @pl.when(i >= 2)
            def _():
                pltpu.make_async_copy(
                    sbufs[b], agg.at[rowv.at[i - 2]], ssems[b]).wait()

            pltpu.make_async_copy(
                hs_hbm.at[c].at[colv.at[i]], gbufs[b], gsems[b]).wait()

            @pl.loop(0, G)
            def _edge(e):
                vv = plsc.load_gather(
                    valv, [jnp.full((L,), i, jnp.int32),
                           jnp.full((L,), e, jnp.int32)])
                for j in range(0, DH, L):
                    sbufs[b][e, pl.ds(j, L)] = gbufs[b][e, pl.ds(j, L)] * vv

            pltpu.async_copy(sbufs[b], agg.at[rowv.at[i]], ssems[b], add=True)

            @pl.when(i + 2 < NCH)
            def _():
                pltpu.async_copy(
                    hs_hbm.at[c].at[colv.at[i + 2]], gbufs[b], gsems[b])

    pltpu.make_async_copy(sbuf0, agg.at[rowv.at[NCH - 2]], ssem0).wait()
    pltpu.make_async_copy(sbuf1, agg.at[rowv.at[NCH - 1]], ssem1).wait()

    plsc.subcore_barrier()
    pltpu.sync_copy(agg.at[pl.ds(s * RPT, RPT)],
                    out_hbm.at[c, pl.ds(s * RPT, RPT)])


def kernel(x, adj_indices, adj_values, W1, b1, W2, b2):
    h = _mlp(x, W1, b1, W2, b2)
    hs = jnp.zeros((NC, NP, DH), jnp.float32)
    hs = hs.at[:, :N, :].set(
        jnp.transpose(h.reshape(N, NC, DH), (1, 0, 2)))
    row = adj_indices[0].reshape(NS, NCH, G)
    col = adj_indices[1].reshape(NS, NCH, G)
    vals = adj_values.reshape(NS, NCH, G)
    zeros = jnp.zeros((NP, DH), jnp.float32)
    for _ in range(K):
        part = _sc_prop(hs, row, col, vals, zeros)
        hs = _blend(part, hs)
    return jnp.transpose(hs[:, :N, :], (1, 0, 2)).reshape(N, D)


# speedup vs baseline: 8.8307x; 2.3845x over previous
import dataclasses
import functools

import jax
import jax.numpy as jnp
from jax import lax
from jax.experimental import pallas as pl
from jax.experimental.pallas import tpu as pltpu
from jax.experimental.pallas import tpu_sc as plsc

ALPHA = 0.01
K = 10

N = 10000
E = 320000
D = 128

NC = 2
NS = 16
L = 16
DH = D // NC

NP = 10240
G = 80
EPT = E // NS
NCH = EPT // G
RPT = NP // NS

_vector_mesh = plsc.VectorSubcoreMesh(core_axis_name="c", subcore_axis_name="s")

_sc_params = pltpu.CompilerParams()
for _f, _v in (("needs_layout_passes", False), ("use_tc_tiling_on_sc", False)):
    if _f in pltpu.CompilerParams.__dataclass_fields__:
        _sc_params = dataclasses.replace(_sc_params, **{_f: _v})


def _mlp_body(x_ref, w1_ref, b1_ref, w2_ref, b2_ref, o_ref):
    h = lax.dot_general(x_ref[...], w1_ref[...], (((1,), (1,)), ((), ())),
                        preferred_element_type=jnp.float32,
                        precision=lax.Precision.HIGHEST)
    h = jnp.maximum(h + b1_ref[...], 0.0)
    o = lax.dot_general(h, w2_ref[...], (((1,), (1,)), ((), ())),
                        preferred_element_type=jnp.float32,
                        precision=lax.Precision.HIGHEST)
    o_ref[...] = o + b2_ref[...]


def _mlp(x, W1, b1, W2, b2):
    bm = 2000
    return pl.pallas_call(
        _mlp_body,
        grid=(N // bm,),
        in_specs=[
            pl.BlockSpec((bm, D), lambda i: (i, 0)),
            pl.BlockSpec((D, D), lambda i: (0, 0)),
            pl.BlockSpec((1, D), lambda i: (0, 0)),
            pl.BlockSpec((D, D), lambda i: (0, 0)),
            pl.BlockSpec((1, D), lambda i: (0, 0)),
        ],
        out_specs=pl.BlockSpec((bm, D), lambda i: (i, 0)),
        out_shape=jax.ShapeDtypeStruct((N, D), jnp.float32),
    )(x, W1, b1.reshape(1, D), W2, b2.reshape(1, D))


def _blend_body(p_ref, h_ref, o_ref):
    o_ref[...] = (1.0 - ALPHA) * p_ref[...] + ALPHA * h_ref[...]


def _blend(part, hs):
    bm = 2048
    spec = pl.BlockSpec((NC, bm, DH), lambda i: (0, i, 0))
    return pl.pallas_call(
        _blend_body,
        grid=(NP // bm,),
        in_specs=[spec, spec],
        out_specs=spec,
        out_shape=jax.ShapeDtypeStruct((NC, NP, DH), jnp.float32),
    )(part, hs)


@functools.partial(
    pl.kernel,
    mesh=_vector_mesh,
    out_type=jax.ShapeDtypeStruct((NC, NP, DH), jnp.float32),
    scratch_types=[
        pltpu.VMEM((NCH, G), jnp.int32),
        pltpu.VMEM((NCH, G), jnp.int32),
        pltpu.VMEM((NCH, G), jnp.float32),
        pltpu.VMEM((G, DH), jnp.float32),
        pltpu.VMEM((G, DH), jnp.float32),
        pltpu.VMEM((G, DH), jnp.float32),
        pltpu.VMEM((G, DH), jnp.float32),
        pltpu.VMEM_SHARED((NP, DH), jnp.float32),
        pltpu.SemaphoreType.DMA,
        pltpu.SemaphoreType.DMA,
        pltpu.SemaphoreType.DMA,
        pltpu.SemaphoreType.DMA,
    ],
    compiler_params=_sc_params,
)
def _sc_prop(hs_hbm, row_hbm, col_hbm, val_hbm, zero_hbm, out_hbm,
             rowv, colv, valv, gbuf0, gbuf1, sbuf0, sbuf1, agg,
             gsem0, gsem1, ssem0, ssem1):
    c = lax.axis_index("c")
    s = lax.axis_index("s")
    gbufs = (gbuf0, gbuf1)
    sbufs = (sbuf0, sbuf1)
    gsems = (gsem0, gsem1)
    ssems = (ssem0, ssem1)

    pltpu.sync_copy(row_hbm.at[s], rowv)
    pltpu.sync_copy(col_hbm.at[s], colv)
    pltpu.sync_copy(val_hbm.at[s], valv)
    pltpu.sync_copy(zero_hbm.at[pl.ds(s * RPT, RPT)],
                    agg.at[pl.ds(s * RPT, RPT)])
    plsc.subcore_barrier()

    pltpu.async_copy(hs_hbm.at[c].at[colv.at[0]], gbuf0, gsem0)
    pltpu.async_copy(hs_hbm.at[c].at[colv.at[1]], gbuf1, gsem1)

    @pl.loop(0, NCH // 2)
    def _pair(t):
        for b in range(2):
            i = 2 * t + b

            @pl.when(i >= 2)
            def _():
                pltpu.make_async_copy(
                    sbufs[b], agg.at[rowv.at[i - 2]], ssems[b]).wait()

            pltpu.make_async_copy(
                hs_hbm.at[c].at[colv.at[i]], gbufs[b], gsems[b]).wait()

            @plsc.parallel_loop(0, G, step=L, unroll=2)
            def _grp(e0):
                vv16 = valv[i, pl.ds(e0, L)]
                for k in range(L):
                    vv = vv16[k]
                    for j in range(0, DH, L):
                        sbufs[b][e0 + k, pl.ds(j, L)] = (
                            gbufs[b][e0 + k, pl.ds(j, L)] * vv)

            pltpu.async_copy(sbufs[b], agg.at[rowv.at[i]], ssems[b], add=True)

            @pl.when(i + 2 < NCH)
            def _():
                pltpu.async_copy(
                    hs_hbm.at[c].at[colv.at[i + 2]], gbufs[b], gsems[b])

    pltpu.make_async_copy(sbuf0, agg.at[rowv.at[NCH - 2]], ssem0).wait()
    pltpu.make_async_copy(sbuf1, agg.at[rowv.at[NCH - 1]], ssem1).wait()

    plsc.subcore_barrier()
    pltpu.sync_copy(agg.at[pl.ds(s * RPT, RPT)],
                    out_hbm.at[c, pl.ds(s * RPT, RPT)])


def kernel(x, adj_indices, adj_values, W1, b1, W2, b2):
    h = _mlp(x, W1, b1, W2, b2)
    hs = jnp.zeros((NC, NP, DH), jnp.float32)
    hs = hs.at[:, :N, :].set(
        jnp.transpose(h.reshape(N, NC, DH), (1, 0, 2)))
    row = adj_indices[0].reshape(NS, NCH, G)
    col = adj_indices[1].reshape(NS, NCH, G)
    vals = adj_values.reshape(NS, NCH, G)
    zeros = jnp.zeros((NP, DH), jnp.float32)
    for _ in range(K):
        part = _sc_prop(hs, row, col, vals, zeros)
        hs = _blend(part, hs)
    return jnp.transpose(hs[:, :N, :], (1, 0, 2)).reshape(N, D)


# speedup vs baseline: 9.7379x; 1.1027x over previous
import dataclasses
import functools

import jax
import jax.numpy as jnp
from jax import lax
from jax.experimental import pallas as pl
from jax.experimental.pallas import tpu as pltpu
from jax.experimental.pallas import tpu_sc as plsc

ALPHA = 0.01
K = 10

N = 10000
E = 320000
D = 128

NC = 2
NS = 16
L = 16
DH = D // NC

NP = 10240
G = 80
EPT = E // NS
NCH = EPT // G
RPT = NP // NS

_vector_mesh = plsc.VectorSubcoreMesh(core_axis_name="c", subcore_axis_name="s")

_sc_params = pltpu.CompilerParams()
for _f, _v in (("needs_layout_passes", False), ("use_tc_tiling_on_sc", False)):
    if _f in pltpu.CompilerParams.__dataclass_fields__:
        _sc_params = dataclasses.replace(_sc_params, **{_f: _v})


def _mlp_body(x_ref, w1_ref, b1_ref, w2_ref, b2_ref, o_ref):
    h = lax.dot_general(x_ref[...], w1_ref[...], (((1,), (1,)), ((), ())),
                        preferred_element_type=jnp.float32,
                        precision=lax.Precision.HIGHEST)
    h = jnp.maximum(h + b1_ref[...], 0.0)
    o = lax.dot_general(h, w2_ref[...], (((1,), (1,)), ((), ())),
                        preferred_element_type=jnp.float32,
                        precision=lax.Precision.HIGHEST)
    o_ref[...] = o + b2_ref[...]


def _mlp(x, W1, b1, W2, b2):
    bm = 2000
    return pl.pallas_call(
        _mlp_body,
        grid=(N // bm,),
        in_specs=[
            pl.BlockSpec((bm, D), lambda i: (i, 0)),
            pl.BlockSpec((D, D), lambda i: (0, 0)),
            pl.BlockSpec((1, D), lambda i: (0, 0)),
            pl.BlockSpec((D, D), lambda i: (0, 0)),
            pl.BlockSpec((1, D), lambda i: (0, 0)),
        ],
        out_specs=pl.BlockSpec((bm, D), lambda i: (i, 0)),
        out_shape=jax.ShapeDtypeStruct((N, D), jnp.float32),
    )(x, W1, b1.reshape(1, D), W2, b2.reshape(1, D))


@functools.partial(
    pl.kernel,
    mesh=_vector_mesh,
    out_type=jax.ShapeDtypeStruct((NC, NP, DH), jnp.float32),
    scratch_types=[
        pltpu.VMEM((NCH, G), jnp.int32),
        pltpu.VMEM((NCH, G), jnp.int32),
        pltpu.VMEM((NCH, G), jnp.float32),
        pltpu.VMEM((G, DH), jnp.float32),
        pltpu.VMEM((G, DH), jnp.float32),
        pltpu.VMEM((G, DH), jnp.float32),
        pltpu.VMEM((G, DH), jnp.float32),
        pltpu.VMEM_SHARED((NP, DH), jnp.float32),
        pltpu.SemaphoreType.DMA,
        pltpu.SemaphoreType.DMA,
        pltpu.SemaphoreType.DMA,
        pltpu.SemaphoreType.DMA,
    ],
    compiler_params=_sc_params,
)
def _sc_prop(hs_hbm, row_hbm, col_hbm, val_hbm, out_hbm,
             rowv, colv, valv, gbuf0, gbuf1, sbuf0, sbuf1, agg,
             gsem0, gsem1, ssem0, ssem1):
    c = lax.axis_index("c")
    s = lax.axis_index("s")
    gbufs = (gbuf0, gbuf1)
    sbufs = (sbuf0, sbuf1)
    gsems = (gsem0, gsem1)
    ssems = (ssem0, ssem1)

    pltpu.sync_copy(row_hbm.at[s], rowv)
    pltpu.sync_copy(col_hbm.at[s], colv)
    pltpu.sync_copy(val_hbm.at[s], valv)

    @plsc.parallel_loop(0, G, unroll=4)
    def _z(e):
        for j in range(0, DH, L):
            gbuf0[e, pl.ds(j, L)] = jnp.zeros((L,), jnp.float32)

    @pl.loop(0, RPT // G)
    def _zc(r):
        pltpu.sync_copy(gbuf0, agg.at[pl.ds(s * RPT + r * G, G)])

    plsc.subcore_barrier()

    pltpu.async_copy(hs_hbm.at[c].at[colv.at[0]], gbuf0, gsem0)
    pltpu.async_copy(hs_hbm.at[c].at[colv.at[1]], gbuf1, gsem1)

    @pl.loop(0, NCH // 2)
    def _pair(t):
        for b in range(2):
            i = 2 * t + b

            @pl.when(i >= 2)
            def _():
                pltpu.make_async_copy(
                    sbufs[b], agg.at[rowv.at[i - 2]], ssems[b]).wait()

            pltpu.make_async_copy(
                hs_hbm.at[c].at[colv.at[i]], gbufs[b], gsems[b]).wait()

            @plsc.parallel_loop(0, G, step=L, unroll=2)
            def _grp(e0):
                vv16 = valv[i, pl.ds(e0, L)]
                for k in range(L):
                    vv = vv16[k]
                    for j in range(0, DH, L):
                        sbufs[b][e0 + k, pl.ds(j, L)] = (
                            gbufs[b][e0 + k, pl.ds(j, L)] * vv)

            pltpu.async_copy(sbufs[b], agg.at[rowv.at[i]], ssems[b], add=True)

            @pl.when(i + 2 < NCH)
            def _():
                pltpu.async_copy(
                    hs_hbm.at[c].at[colv.at[i + 2]], gbufs[b], gsems[b])

    pltpu.make_async_copy(sbuf0, agg.at[rowv.at[NCH - 2]], ssem0).wait()
    pltpu.make_async_copy(sbuf1, agg.at[rowv.at[NCH - 1]], ssem1).wait()

    plsc.subcore_barrier()

    @pl.loop(0, RPT // G)
    def _blend(r):
        off = s * RPT + r * G
        pltpu.sync_copy(hs_hbm.at[c].at[pl.ds(off, G)], gbuf0)
        pltpu.sync_copy(agg.at[pl.ds(off, G)], gbuf1)

        @plsc.parallel_loop(0, G, unroll=4)
        def _b(e):
            for j in range(0, DH, L):
                sbuf0[e, pl.ds(j, L)] = (
                    (1.0 - ALPHA) * gbuf1[e, pl.ds(j, L)]
                    + ALPHA * gbuf0[e, pl.ds(j, L)])

        pltpu.sync_copy(sbuf0, out_hbm.at[c, pl.ds(off, G)])


def kernel(x, adj_indices, adj_values, W1, b1, W2, b2):
    h = _mlp(x, W1, b1, W2, b2)
    hs = jnp.zeros((NC, NP, DH), jnp.float32)
    hs = hs.at[:, :N, :].set(
        jnp.transpose(h.reshape(N, NC, DH), (1, 0, 2)))
    row = adj_indices[0].reshape(NS, NCH, G)
    col = adj_indices[1].reshape(NS, NCH, G)
    vals = adj_values.reshape(NS, NCH, G)
    for _ in range(K):
        hs = _sc_prop(hs, row, col, vals)
    return jnp.transpose(hs[:, :N, :], (1, 0, 2)).reshape(N, D)
